# baseline (speedup 1.0000x reference)
import functools
import math

import jax
import jax.numpy as jnp
from jax import lax
from jax.experimental import pallas as pl
from jax.experimental.pallas import tpu as pltpu
from jax.experimental.pallas import tpu_sc as plsc

NC = 2
NS = 16
NW = NC * NS
CH = 128


def _qkv_body(h_ref, wq_ref, wkv_ref, q_ref, kv_ref):
    hb = h_ref[...]
    q_ref[...] = jnp.dot(hb, wq_ref[...], preferred_element_type=jnp.float32)
    kv_ref[...] = jnp.dot(
        hb, wkv_ref[...],
        preferred_element_type=jnp.float32).astype(jnp.bfloat16)


def _edge_body(E, BE, D, H, qd_ref, kvs_ref, ef_ref, d2_ref, w1_ref, b1_ref,
               w2_ref, b2_ref, esum_ref, erep_ref, s1_ref, s2_ref,
               num_ref, srow_ref):
    i = pl.program_id(0)
    qd = qd_ref[...]
    kv = kvs_ref[...].astype(jnp.float32)
    ks = kv[:, :D]
    vs = kv[:, D:]
    hid = jnp.maximum(
        jnp.dot(ef_ref[...], w1_ref[...], preferred_element_type=jnp.float32)
        + b1_ref[...], 0.0)
    bias = jnp.dot(hid, w2_ref[...], preferred_element_type=jnp.float32) \
        + b2_ref[...]
    score = jnp.dot(qd * ks, esum_ref[...],
                    preferred_element_type=jnp.float32)
    ex = jnp.exp(score + bias)
    row = i * BE + lax.broadcasted_iota(jnp.int32, (BE, 1), 0)
    ex = ex * (row < E).astype(jnp.float32)
    exrep = jnp.dot(ex, erep_ref[...],
                    preferred_element_type=jnp.float32)
    num_ref[...] = exrep * vs
    m16 = jnp.bitwise_and(d2_ref[...], 15)
    oh = (m16 == lax.broadcasted_iota(jnp.int32, (BE, 16), 1))
    oh = oh.astype(jnp.float32)
    srow_ref[...] = (
        jnp.dot(oh, s1_ref[...], preferred_element_type=jnp.float32)
        * jnp.dot(ex, s2_ref[...], preferred_element_type=jnp.float32))


def _gather_body(KJ, q_hbm, kv_hbm, dstr, srcr, qd_hbm, kvs_hbm,
                 dloc, sloc, qbuf0, kvbuf0, qbuf1, kvbuf1,
                 semg0, semg1, semw0, semw1):
    cid = lax.axis_index("c")
    sid = lax.axis_index("s")
    wid = sid * NC + cid
    pltpu.sync_copy(dstr.at[wid], dloc)
    pltpu.sync_copy(srcr.at[wid], sloc)
    qbufs = (qbuf0, qbuf1)
    kvbufs = (kvbuf0, kvbuf1)
    semgs = (semg0, semg1)
    semws = (semw0, semw1)

    pltpu.async_copy(q_hbm.at[dloc.at[0]], qbuf0, semg0)
    pltpu.async_copy(kv_hbm.at[sloc.at[0]], kvbuf0, semg0)

    def outer(jj, carry):
        for b in range(2):
            j = jj * 2 + b
            p = b
            o = 1 - b

            @pl.when(j < KJ)
            def _():
                base = (wid * KJ + j) * CH
                @pl.when(j >= 1)
                def _():
                    pltpu.make_async_copy(
                        qbufs[o], qd_hbm.at[pl.ds(base, CH)], semws[o]).wait()
                    pltpu.make_async_copy(
                        kvbufs[o], kvs_hbm.at[pl.ds(base, CH)],
                        semws[o]).wait()

                @pl.when(j + 1 < KJ)
                def _():
                    pltpu.async_copy(q_hbm.at[dloc.at[j + 1]], qbufs[o],
                                     semgs[o])
                    pltpu.async_copy(kv_hbm.at[sloc.at[j + 1]], kvbufs[o],
                                     semgs[o])

                pltpu.make_async_copy(q_hbm.at[dloc.at[j]], qbufs[p],
                                      semgs[p]).wait()
                pltpu.make_async_copy(kv_hbm.at[sloc.at[j]], kvbufs[p],
                                      semgs[p]).wait()

                @pl.when(j == KJ - 1)
                def _():
                    pltpu.sync_copy(qbufs[p], qd_hbm.at[pl.ds(base, CH)])
                    pltpu.sync_copy(kvbufs[p], kvs_hbm.at[pl.ds(base, CH)])

                @pl.when(j < KJ - 1)
                def _():
                    pltpu.async_copy(qbufs[p], qd_hbm.at[pl.ds(base, CH)],
                                     semws[p])
                    pltpu.async_copy(kvbufs[p], kvs_hbm.at[pl.ds(base, CH)],
                                     semws[p])
        return carry

    lax.fori_loop(0, (KJ + 1) // 2, outer, 0)


def _scatter_body(KJ, ZR, ZR16, num_hbm, srow_hbm, dstr, zeros_hbm,
                  accv_hbm, accs_hbm, shared, shared_s, dloc, d16buf,
                  cbuf, sbuf):
    cid = lax.axis_index("c")
    sid = lax.axis_index("s")
    wid = sid * NC + cid
    pltpu.sync_copy(zeros_hbm, shared.at[pl.ds(sid * ZR, ZR)])
    pltpu.sync_copy(zeros_hbm.at[pl.ds(0, ZR16)],
                    shared_s.at[pl.ds(sid * ZR16, ZR16)])
    plsc.subcore_barrier()
    pltpu.sync_copy(dstr.at[wid], dloc)

    def step(j, carry):
        base = (wid * KJ + j) * CH
        for g in range(CH // 16):
            dvec = dloc[j, pl.ds(g * 16, 16)]
            d16buf[pl.ds(g * 16, 16)] = lax.shift_right_logical(dvec, 4)
        pltpu.sync_copy(num_hbm.at[pl.ds(base, CH)], cbuf)
        pltpu.sync_copy(srow_hbm.at[pl.ds(base, CH)], sbuf)
        pltpu.sync_copy(cbuf, shared.at[dloc.at[j]], add=True)
        pltpu.sync_copy(sbuf, shared_s.at[d16buf], add=True)
        return carry

    lax.fori_loop(0, KJ, step, 0)
    plsc.subcore_barrier()
    pltpu.sync_copy(shared.at[pl.ds(sid * ZR, ZR)],
                    accv_hbm.at[cid, pl.ds(sid * ZR, ZR)])
    pltpu.sync_copy(shared_s.at[pl.ds(sid * ZR16, ZR16)],
                    accs_hbm.at[cid, pl.ds(sid * ZR16, ZR16)])


def _final_body(D, accv_ref, s_ref, h_ref, ow_ref, ob_ref, erep_ref,
                pa_ref, out_ref):
    ovec = accv_ref[0] + accv_ref[1]
    s8 = s_ref[0] + s_ref[1]
    srep = jnp.dot(s8, erep_ref[...], preferred_element_type=jnp.float32)
    att = ovec / jnp.maximum(srep, 1e-20)
    y = jnp.dot(att, ow_ref[...], preferred_element_type=jnp.float32) \
        + ob_ref[...] + h_ref[...]
    a = pa_ref[0, 0]
    out_ref[...] = jnp.where(y >= 0, y, a * y)


def kernel(h, edge_index, edge_feat, W_q, W_k, W_v, eW1, eb1, eW2, eb2,
           out_W, out_b, prelu_a):
    N, D = h.shape
    E = edge_index.shape[1]
    DE = edge_feat.shape[1]
    H = eW2.shape[1]
    hd = D // H
    f32 = jnp.float32

    Wq = W_q * (1.0 / math.sqrt(hd))
    Wkv = jnp.concatenate([W_k, W_v], axis=1)
    eye = jnp.eye(H, dtype=f32)
    erep = jnp.repeat(eye, hd, axis=1)
    esum = erep.T
    s1 = jnp.repeat(jnp.eye(16, dtype=f32), H, axis=1)
    s2 = jnp.tile(eye, (1, 16))
    b1 = eb1.reshape(1, D)
    b2 = eb2.reshape(1, H)
    ob = out_b.reshape(1, D)
    pa = prelu_a.reshape(1, 1)

    KJ = -(-E // (NW * CH))
    Epad = NW * CH * KJ
    src = edge_index[0]
    dst = edge_index[1]
    pad = Epad - E
    dstp = jnp.concatenate([dst, jnp.zeros((pad,), jnp.int32)])
    srcp = jnp.concatenate([src, jnp.zeros((pad,), jnp.int32)])
    dstr = dstp.reshape(NW, KJ, CH)
    srcr = srcp.reshape(NW, KJ, CH)
    dst2 = dstp.reshape(Epad, 1)
    efp = jnp.concatenate([edge_feat, jnp.zeros((pad, DE), f32)])
    Np = -(-N // (NS * 8)) * (NS * 8)
    ZR = Np // NS
    NP16 = -(-(-(-Np // 16)) // (NS * 8)) * (NS * 8)
    ZR16 = NP16 // NS
    zeros = jnp.zeros((ZR, D), f32)

    BN = 1000 if N % 1000 == 0 else N
    q, kv = pl.pallas_call(
        _qkv_body,
        grid=(N // BN,),
        in_specs=[
            pl.BlockSpec((BN, D), lambda i: (i, 0)),
            pl.BlockSpec((D, D), lambda i: (0, 0)),
            pl.BlockSpec((D, 2 * D), lambda i: (0, 0)),
        ],
        out_specs=[
            pl.BlockSpec((BN, D), lambda i: (i, 0)),
            pl.BlockSpec((BN, 2 * D), lambda i: (i, 0)),
        ],
        out_shape=[
            jax.ShapeDtypeStruct((N, D), f32),
            jax.ShapeDtypeStruct((N, 2 * D), jnp.bfloat16),
        ],
    )(h, Wq, Wkv)
    kvp = lax.bitcast_convert_type(kv.reshape(N, D, 2), f32)

    mesh = plsc.VectorSubcoreMesh(core_axis_name="c", subcore_axis_name="s")
    gather = pl.kernel(
        functools.partial(_gather_body, KJ),
        out_type=(
            jax.ShapeDtypeStruct((Epad, D), f32),
            jax.ShapeDtypeStruct((Epad, D), f32),
        ),
        mesh=mesh,
        scratch_types=(
            pltpu.VMEM((KJ, CH), jnp.int32),
            pltpu.VMEM((KJ, CH), jnp.int32),
            pltpu.VMEM((CH, D), f32),
            pltpu.VMEM((CH, D), f32),
            pltpu.VMEM((CH, D), f32),
            pltpu.VMEM((CH, D), f32),
            pltpu.SemaphoreType.DMA,
            pltpu.SemaphoreType.DMA,
            pltpu.SemaphoreType.DMA,
            pltpu.SemaphoreType.DMA,
        ),
    )
    qd, kvsp = gather(q, kvp, dstr, srcr)
    kvs = lax.bitcast_convert_type(kvsp, jnp.bfloat16).reshape(Epad, 2 * D)

    BE = 1024
    nbe = -(-Epad // BE)
    num, srow = pl.pallas_call(
        functools.partial(_edge_body, E, BE, D, H),
        grid=(nbe,),
        in_specs=[
            pl.BlockSpec((BE, D), lambda i: (i, 0)),
            pl.BlockSpec((BE, 2 * D), lambda i: (i, 0)),
            pl.BlockSpec((BE, DE), lambda i: (i, 0)),
            pl.BlockSpec((BE, 1), lambda i: (i, 0)),
            pl.BlockSpec((DE, D), lambda i: (0, 0)),
            pl.BlockSpec((1, D), lambda i: (0, 0)),
            pl.BlockSpec((D, H), lambda i: (0, 0)),
            pl.BlockSpec((1, H), lambda i: (0, 0)),
            pl.BlockSpec((D, H), lambda i: (0, 0)),
            pl.BlockSpec((H, D), lambda i: (0, 0)),
            pl.BlockSpec((16, D), lambda i: (0, 0)),
            pl.BlockSpec((H, D), lambda i: (0, 0)),
        ],
        out_specs=[
            pl.BlockSpec((BE, D), lambda i: (i, 0)),
            pl.BlockSpec((BE, D), lambda i: (i, 0)),
        ],
        out_shape=[
            jax.ShapeDtypeStruct((Epad, D), f32),
            jax.ShapeDtypeStruct((Epad, D), f32),
        ],
    )(qd, kvs, efp, dst2, eW1, b1, eW2, b2, esum, erep, s1, s2)

    scatter = pl.kernel(
        functools.partial(_scatter_body, KJ, ZR, ZR16),
        out_type=(
            jax.ShapeDtypeStruct((NC, Np, D), f32),
            jax.ShapeDtypeStruct((NC, NP16, D), f32),
        ),
        mesh=mesh,
        scratch_types=(
            pltpu.VMEM_SHARED((Np, D), f32),
            pltpu.VMEM_SHARED((NP16, D), f32),
            pltpu.VMEM((KJ, CH), jnp.int32),
            pltpu.VMEM((CH,), jnp.int32),
            pltpu.VMEM((CH, D), f32),
            pltpu.VMEM((CH, D), f32),
        ),
    )
    accv, accs = scatter(num, srow, dstr, zeros)
    s3 = accs.reshape(NC, NP16 * 16, H)

    h_out = pl.pallas_call(
        functools.partial(_final_body, D),
        grid=(N // BN,),
        in_specs=[
            pl.BlockSpec((NC, BN, D), lambda i: (0, i, 0)),
            pl.BlockSpec((NC, BN, H), lambda i: (0, i, 0)),
            pl.BlockSpec((BN, D), lambda i: (i, 0)),
            pl.BlockSpec((D, D), lambda i: (0, 0)),
            pl.BlockSpec((1, D), lambda i: (0, 0)),
            pl.BlockSpec((H, D), lambda i: (0, 0)),
            pl.BlockSpec((1, 1), lambda i: (0, 0)),
        ],
        out_specs=pl.BlockSpec((BN, D), lambda i: (i, 0)),
        out_shape=jax.ShapeDtypeStruct((N, D), f32),
    )(accv, s3, h, out_W, ob, erep, pa)
    return h_out


# speedup vs baseline: 1.8607x; 1.8607x over previous
import functools
import math

import jax
import jax.numpy as jnp
from jax import lax
from jax.experimental import pallas as pl
from jax.experimental.pallas import tpu as pltpu
from jax.experimental.pallas import tpu_sc as plsc

NC = 2
NS = 16
NW = NC * NS
CH = 128


def _qkv_body(D, h_ref, wq_ref, wkv_ref, q_ref, kv_ref):
    hb = h_ref[...]
    q_ref[...] = jnp.dot(hb, wq_ref[...], preferred_element_type=jnp.float32)
    kvf = jnp.dot(hb, wkv_ref[...], preferred_element_type=jnp.float32)
    n = kvf.shape[0]
    k = kvf[:, :D].astype(jnp.bfloat16)
    v = kvf[:, D:].astype(jnp.bfloat16)
    kvi = jnp.concatenate([k[:, None, :], v[:, None, :]],
                          axis=1).reshape(2 * n, D)
    kv_ref.bitcast(jnp.bfloat16)[...] = kvi


def _edge_body(E, BE, D, H, qd_ref, kvs_ref, ef_ref, d2_ref, w1_ref, b1_ref,
               w2_ref, b2_ref, esum_ref, erep_ref, s1_ref, s2_ref,
               num_ref, srow_ref):
    i = pl.program_id(0)
    qd = qd_ref[...]
    kvv = kvs_ref.bitcast(jnp.bfloat16)[...]
    kv3 = kvv.reshape(BE, 2, D)
    ks = kv3[:, 0, :].astype(jnp.float32)
    vs = kv3[:, 1, :].astype(jnp.float32)
    hid = jnp.maximum(
        jnp.dot(ef_ref[...], w1_ref[...], preferred_element_type=jnp.float32)
        + b1_ref[...], 0.0)
    bias = jnp.dot(hid, w2_ref[...], preferred_element_type=jnp.float32) \
        + b2_ref[...]
    score = jnp.dot(qd * ks, esum_ref[...],
                    preferred_element_type=jnp.float32)
    ex = jnp.exp(score + bias)
    row = i * BE + lax.broadcasted_iota(jnp.int32, (BE, 1), 0)
    ex = ex * (row < E).astype(jnp.float32)
    exrep = jnp.dot(ex, erep_ref[...],
                    preferred_element_type=jnp.float32)
    num_ref[...] = exrep * vs
    m16 = jnp.bitwise_and(d2_ref[...], 15)
    oh = (m16 == lax.broadcasted_iota(jnp.int32, (BE, 16), 1))
    oh = oh.astype(jnp.float32)
    srow_ref[...] = (
        jnp.dot(oh, s1_ref[...], preferred_element_type=jnp.float32)
        * jnp.dot(ex, s2_ref[...], preferred_element_type=jnp.float32))


def _gather_body(KJ, q_hbm, kv_hbm, dstr, srcr, qd_hbm, kvs_hbm,
                 dloc, sloc, qbuf0, kvbuf0, qbuf1, kvbuf1,
                 semg0, semg1, semw0, semw1):
    cid = lax.axis_index("c")
    sid = lax.axis_index("s")
    wid = sid * NC + cid
    pltpu.sync_copy(dstr.at[wid], dloc)
    pltpu.sync_copy(srcr.at[wid], sloc)
    qbufs = (qbuf0, qbuf1)
    kvbufs = (kvbuf0, kvbuf1)
    semgs = (semg0, semg1)
    semws = (semw0, semw1)

    pltpu.async_copy(q_hbm.at[dloc.at[0]], qbuf0, semg0)
    pltpu.async_copy(kv_hbm.at[sloc.at[0]], kvbuf0, semg0)

    def outer(jj, carry):
        for b in range(2):
            j = jj * 2 + b
            p = b
            o = 1 - b

            @pl.when(j < KJ)
            def _():
                base = (wid * KJ + j) * CH
                @pl.when(j >= 1)
                def _():
                    pltpu.make_async_copy(
                        qbufs[o], qd_hbm.at[pl.ds(base, CH)], semws[o]).wait()
                    pltpu.make_async_copy(
                        kvbufs[o], kvs_hbm.at[pl.ds(base, CH)],
                        semws[o]).wait()

                @pl.when(j + 1 < KJ)
                def _():
                    pltpu.async_copy(q_hbm.at[dloc.at[j + 1]], qbufs[o],
                                     semgs[o])
                    pltpu.async_copy(kv_hbm.at[sloc.at[j + 1]], kvbufs[o],
                                     semgs[o])

                pltpu.make_async_copy(q_hbm.at[dloc.at[j]], qbufs[p],
                                      semgs[p]).wait()
                pltpu.make_async_copy(kv_hbm.at[sloc.at[j]], kvbufs[p],
                                      semgs[p]).wait()

                @pl.when(j == KJ - 1)
                def _():
                    pltpu.sync_copy(qbufs[p], qd_hbm.at[pl.ds(base, CH)])
                    pltpu.sync_copy(kvbufs[p], kvs_hbm.at[pl.ds(base, CH)])

                @pl.when(j < KJ - 1)
                def _():
                    pltpu.async_copy(qbufs[p], qd_hbm.at[pl.ds(base, CH)],
                                     semws[p])
                    pltpu.async_copy(kvbufs[p], kvs_hbm.at[pl.ds(base, CH)],
                                     semws[p])
        return carry

    lax.fori_loop(0, (KJ + 1) // 2, outer, 0)


def _scatter_body(KJ, ZR, ZR16, num_hbm, srow_hbm, dstr, zeros_hbm,
                  accv_hbm, accs_hbm, shared, shared_s, dloc, d16buf,
                  cbuf, sbuf):
    cid = lax.axis_index("c")
    sid = lax.axis_index("s")
    wid = sid * NC + cid
    pltpu.sync_copy(zeros_hbm, shared.at[pl.ds(sid * ZR, ZR)])
    pltpu.sync_copy(zeros_hbm.at[pl.ds(0, ZR16)],
                    shared_s.at[pl.ds(sid * ZR16, ZR16)])
    plsc.subcore_barrier()
    pltpu.sync_copy(dstr.at[wid], dloc)

    def step(j, carry):
        base = (wid * KJ + j) * CH
        for g in range(CH // 16):
            dvec = dloc[j, pl.ds(g * 16, 16)]
            d16buf[pl.ds(g * 16, 16)] = lax.shift_right_logical(dvec, 4)
        pltpu.sync_copy(num_hbm.at[pl.ds(base, CH)], cbuf)
        pltpu.sync_copy(srow_hbm.at[pl.ds(base, CH)], sbuf)
        pltpu.sync_copy(cbuf, shared.at[dloc.at[j]], add=True)
        pltpu.sync_copy(sbuf, shared_s.at[d16buf], add=True)
        return carry

    lax.fori_loop(0, KJ, step, 0)
    plsc.subcore_barrier()
    pltpu.sync_copy(shared.at[pl.ds(sid * ZR, ZR)],
                    accv_hbm.at[cid, pl.ds(sid * ZR, ZR)])
    pltpu.sync_copy(shared_s.at[pl.ds(sid * ZR16, ZR16)],
                    accs_hbm.at[cid, pl.ds(sid * ZR16, ZR16)])


def _final_body(D, accv_ref, s_ref, h_ref, ow_ref, ob_ref, erep_ref,
                pa_ref, out_ref):
    ovec = accv_ref[0] + accv_ref[1]
    s8 = s_ref[0] + s_ref[1]
    srep = jnp.dot(s8, erep_ref[...], preferred_element_type=jnp.float32)
    att = ovec / jnp.maximum(srep, 1e-20)
    y = jnp.dot(att, ow_ref[...], preferred_element_type=jnp.float32) \
        + ob_ref[...] + h_ref[...]
    a = pa_ref[0, 0]
    out_ref[...] = jnp.where(y >= 0, y, a * y)


def kernel(h, edge_index, edge_feat, W_q, W_k, W_v, eW1, eb1, eW2, eb2,
           out_W, out_b, prelu_a):
    N, D = h.shape
    E = edge_index.shape[1]
    DE = edge_feat.shape[1]
    H = eW2.shape[1]
    hd = D // H
    f32 = jnp.float32

    Wq = W_q * (1.0 / math.sqrt(hd))
    Wkv = jnp.concatenate([W_k, W_v], axis=1)
    eye = jnp.eye(H, dtype=f32)
    erep = jnp.repeat(eye, hd, axis=1)
    esum = erep.T
    s1 = jnp.repeat(jnp.eye(16, dtype=f32), H, axis=1)
    s2 = jnp.tile(eye, (1, 16))
    b1 = eb1.reshape(1, D)
    b2 = eb2.reshape(1, H)
    ob = out_b.reshape(1, D)
    pa = prelu_a.reshape(1, 1)

    KJ = -(-E // (NW * CH))
    Epad = NW * CH * KJ
    src = edge_index[0]
    dst = edge_index[1]
    pad = Epad - E
    dstp = jnp.concatenate([dst, jnp.zeros((pad,), jnp.int32)])
    srcp = jnp.concatenate([src, jnp.zeros((pad,), jnp.int32)])
    dstr = dstp.reshape(NW, KJ, CH)
    srcr = srcp.reshape(NW, KJ, CH)
    dst2 = dstp.reshape(Epad, 1)
    efp = jnp.concatenate([edge_feat, jnp.zeros((pad, DE), f32)])
    Np = -(-N // (NS * 8)) * (NS * 8)
    ZR = Np // NS
    NP16 = -(-(-(-Np // 16)) // (NS * 8)) * (NS * 8)
    ZR16 = NP16 // NS
    zeros = jnp.zeros((ZR, D), f32)

    BN = 1000 if N % 1000 == 0 else N
    q, kvp = pl.pallas_call(
        functools.partial(_qkv_body, D),
        grid=(N // BN,),
        in_specs=[
            pl.BlockSpec((BN, D), lambda i: (i, 0)),
            pl.BlockSpec((D, D), lambda i: (0, 0)),
            pl.BlockSpec((D, 2 * D), lambda i: (0, 0)),
        ],
        out_specs=[
            pl.BlockSpec((BN, D), lambda i: (i, 0)),
            pl.BlockSpec((BN, D), lambda i: (i, 0)),
        ],
        out_shape=[
            jax.ShapeDtypeStruct((N, D), f32),
            jax.ShapeDtypeStruct((N, D), f32),
        ],
    )(h, Wq, Wkv)

    mesh = plsc.VectorSubcoreMesh(core_axis_name="c", subcore_axis_name="s")
    gather = pl.kernel(
        functools.partial(_gather_body, KJ),
        out_type=(
            jax.ShapeDtypeStruct((Epad, D), f32),
            jax.ShapeDtypeStruct((Epad, D), f32),
        ),
        mesh=mesh,
        scratch_types=(
            pltpu.VMEM((KJ, CH), jnp.int32),
            pltpu.VMEM((KJ, CH), jnp.int32),
            pltpu.VMEM((CH, D), f32),
            pltpu.VMEM((CH, D), f32),
            pltpu.VMEM((CH, D), f32),
            pltpu.VMEM((CH, D), f32),
            pltpu.SemaphoreType.DMA,
            pltpu.SemaphoreType.DMA,
            pltpu.SemaphoreType.DMA,
            pltpu.SemaphoreType.DMA,
        ),
    )
    qd, kvs = gather(q, kvp, dstr, srcr)

    BE = 1024
    nbe = -(-Epad // BE)
    num, srow = pl.pallas_call(
        functools.partial(_edge_body, E, BE, D, H),
        grid=(nbe,),
        in_specs=[
            pl.BlockSpec((BE, D), lambda i: (i, 0)),
            pl.BlockSpec((BE, D), lambda i: (i, 0)),
            pl.BlockSpec((BE, DE), lambda i: (i, 0)),
            pl.BlockSpec((BE, 1), lambda i: (i, 0)),
            pl.BlockSpec((DE, D), lambda i: (0, 0)),
            pl.BlockSpec((1, D), lambda i: (0, 0)),
            pl.BlockSpec((D, H), lambda i: (0, 0)),
            pl.BlockSpec((1, H), lambda i: (0, 0)),
            pl.BlockSpec((D, H), lambda i: (0, 0)),
            pl.BlockSpec((H, D), lambda i: (0, 0)),
            pl.BlockSpec((16, D), lambda i: (0, 0)),
            pl.BlockSpec((H, D), lambda i: (0, 0)),
        ],
        out_specs=[
            pl.BlockSpec((BE, D), lambda i: (i, 0)),
            pl.BlockSpec((BE, D), lambda i: (i, 0)),
        ],
        out_shape=[
            jax.ShapeDtypeStruct((Epad, D), f32),
            jax.ShapeDtypeStruct((Epad, D), f32),
        ],
    )(qd, kvs, efp, dst2, eW1, b1, eW2, b2, esum, erep, s1, s2)

    scatter = pl.kernel(
        functools.partial(_scatter_body, KJ, ZR, ZR16),
        out_type=(
            jax.ShapeDtypeStruct((NC, Np, D), f32),
            jax.ShapeDtypeStruct((NC, NP16, D), f32),
        ),
        mesh=mesh,
        scratch_types=(
            pltpu.VMEM_SHARED((Np, D), f32),
            pltpu.VMEM_SHARED((NP16, D), f32),
            pltpu.VMEM((KJ, CH), jnp.int32),
            pltpu.VMEM((CH,), jnp.int32),
            pltpu.VMEM((CH, D), f32),
            pltpu.VMEM((CH, D), f32),
        ),
    )
    accv, accs = scatter(num, srow, dstr, zeros)
    s3 = accs.reshape(NC, NP16 * 16, H)

    h_out = pl.pallas_call(
        functools.partial(_final_body, D),
        grid=(N // BN,),
        in_specs=[
            pl.BlockSpec((NC, BN, D), lambda i: (0, i, 0)),
            pl.BlockSpec((NC, BN, H), lambda i: (0, i, 0)),
            pl.BlockSpec((BN, D), lambda i: (i, 0)),
            pl.BlockSpec((D, D), lambda i: (0, 0)),
            pl.BlockSpec((1, D), lambda i: (0, 0)),
            pl.BlockSpec((H, D), lambda i: (0, 0)),
            pl.BlockSpec((1, 1), lambda i: (0, 0)),
        ],
        out_specs=pl.BlockSpec((BN, D), lambda i: (i, 0)),
        out_shape=jax.ShapeDtypeStruct((N, D), f32),
    )(accv, s3, h, out_W, ob, erep, pa)
    return h_out


# speedup vs baseline: 2.0612x; 1.1078x over previous
import functools
import math

import jax
import jax.numpy as jnp
from jax import lax
from jax.experimental import pallas as pl
from jax.experimental.pallas import tpu as pltpu
from jax.experimental.pallas import tpu_sc as plsc

NC = 2
NS = 16
NW = NC * NS
CH = 128


def _qkv_body(D, h_ref, wq_ref, wkv_ref, q_ref, kv_ref):
    hb = h_ref[...]
    q_ref[...] = jnp.dot(hb, wq_ref[...], preferred_element_type=jnp.float32)
    kvf = jnp.dot(hb, wkv_ref[...], preferred_element_type=jnp.float32)
    n = kvf.shape[0]
    k = kvf[:, :D].astype(jnp.bfloat16)
    v = kvf[:, D:].astype(jnp.bfloat16)
    kvi = jnp.concatenate([k[:, None, :], v[:, None, :]],
                          axis=1).reshape(2 * n, D)
    kv_ref.bitcast(jnp.bfloat16)[...] = kvi


def _edge_body(E, BE, D, H, qd_ref, kvs_ref, ef_ref, d2_ref, w1_ref, b1_ref,
               w2_ref, b2_ref, esum_ref, erep_ref, s1_ref, s2_ref,
               num_ref, srow_ref):
    i = pl.program_id(0)
    qd = qd_ref[...]
    kvv = kvs_ref.bitcast(jnp.bfloat16)[...]
    kv3 = kvv.reshape(BE, 2, D)
    ks = kv3[:, 0, :].astype(jnp.float32)
    vs = kv3[:, 1, :].astype(jnp.float32)
    hid = jnp.maximum(
        jnp.dot(ef_ref[...], w1_ref[...], preferred_element_type=jnp.float32)
        + b1_ref[...], 0.0)
    bias = jnp.dot(hid, w2_ref[...], preferred_element_type=jnp.float32) \
        + b2_ref[...]
    score = jnp.dot(qd * ks, esum_ref[...],
                    preferred_element_type=jnp.float32)
    ex = jnp.exp(score + bias)
    row = i * BE + lax.broadcasted_iota(jnp.int32, (BE, 1), 0)
    ex = ex * (row < E).astype(jnp.float32)
    exrep = jnp.dot(ex, erep_ref[...],
                    preferred_element_type=jnp.float32)
    num_ref[...] = exrep * vs
    m16 = jnp.bitwise_and(d2_ref[...], 15)
    oh = (m16 == lax.broadcasted_iota(jnp.int32, (BE, 16), 1))
    oh = oh.astype(jnp.float32)
    srow_ref[...] = (
        jnp.dot(oh, s1_ref[...], preferred_element_type=jnp.float32)
        * jnp.dot(ex, s2_ref[...], preferred_element_type=jnp.float32))


def _gather_body(KJ, q_hbm, kv_hbm, dstr, srcr, qd_hbm, kvs_hbm,
                 dloc, sloc, qbuf0, kvbuf0, qbuf1, kvbuf1,
                 semg0, semg1, semw0, semw1):
    cid = lax.axis_index("c")
    sid = lax.axis_index("s")
    wid = sid * NC + cid
    pltpu.sync_copy(dstr.at[wid], dloc)
    pltpu.sync_copy(srcr.at[wid], sloc)
    qbufs = (qbuf0, qbuf1)
    kvbufs = (kvbuf0, kvbuf1)
    semgs = (semg0, semg1)
    semws = (semw0, semw1)

    pltpu.async_copy(q_hbm.at[dloc.at[0]], qbuf0, semg0)
    pltpu.async_copy(kv_hbm.at[sloc.at[0]], kvbuf0, semg0)

    def outer(jj, carry):
        for b in range(2):
            j = jj * 2 + b
            p = b
            o = 1 - b

            @pl.when(j < KJ)
            def _():
                base = (wid * KJ + j) * CH
                @pl.when(j >= 1)
                def _():
                    pltpu.make_async_copy(
                        qbufs[o], qd_hbm.at[pl.ds(base, CH)], semws[o]).wait()
                    pltpu.make_async_copy(
                        kvbufs[o], kvs_hbm.at[pl.ds(base, CH)],
                        semws[o]).wait()

                @pl.when(j + 1 < KJ)
                def _():
                    pltpu.async_copy(q_hbm.at[dloc.at[j + 1]], qbufs[o],
                                     semgs[o])
                    pltpu.async_copy(kv_hbm.at[sloc.at[j + 1]], kvbufs[o],
                                     semgs[o])

                pltpu.make_async_copy(q_hbm.at[dloc.at[j]], qbufs[p],
                                      semgs[p]).wait()
                pltpu.make_async_copy(kv_hbm.at[sloc.at[j]], kvbufs[p],
                                      semgs[p]).wait()

                @pl.when(j == KJ - 1)
                def _():
                    pltpu.sync_copy(qbufs[p], qd_hbm.at[pl.ds(base, CH)])
                    pltpu.sync_copy(kvbufs[p], kvs_hbm.at[pl.ds(base, CH)])

                @pl.when(j < KJ - 1)
                def _():
                    pltpu.async_copy(qbufs[p], qd_hbm.at[pl.ds(base, CH)],
                                     semws[p])
                    pltpu.async_copy(kvbufs[p], kvs_hbm.at[pl.ds(base, CH)],
                                     semws[p])
        return carry

    lax.fori_loop(0, (KJ + 1) // 2, outer, 0)


CHS = 64


def _scatter_body(KJ2, ZR, ZR16, num_hbm, srow_hbm, dstr, zeros_hbm,
                  accv_hbm, accs_hbm, shared, shared_s,
                  ibuf0, ibuf1, d16buf0, d16buf1, cbuf0, cbuf1, sbuf0, sbuf1,
                  semr0, semr1, semc0, semc1, sems0, sems1):
    cid = lax.axis_index("c")
    sid = lax.axis_index("s")
    wid = sid * NC + cid
    pltpu.sync_copy(zeros_hbm, shared.at[pl.ds(sid * ZR, ZR)])
    pltpu.sync_copy(zeros_hbm.at[pl.ds(0, ZR16)],
                    shared_s.at[pl.ds(sid * ZR16, ZR16)])
    plsc.subcore_barrier()
    ibufs = (ibuf0, ibuf1)
    d16bufs = (d16buf0, d16buf1)
    cbufs = (cbuf0, cbuf1)
    sbufs = (sbuf0, sbuf1)
    semrs = (semr0, semr1)
    semcs = (semc0, semc1)
    semss = (sems0, sems1)

    base0 = wid * KJ2 * CHS
    pltpu.async_copy(dstr.at[wid, 0], ibuf0, semr0)
    pltpu.async_copy(num_hbm.at[pl.ds(base0, CHS)], cbuf0, semr0)
    pltpu.async_copy(srow_hbm.at[pl.ds(base0, CHS)], sbuf0, semr0)

    def outer(jj, carry):
        for b in range(2):
            j = jj * 2 + b
            p = b
            o = 1 - b

            @pl.when(j < KJ2)
            def _():
                base = (wid * KJ2 + j) * CHS

                @pl.when(j >= 1)
                def _():
                    pltpu.make_async_copy(
                        cbufs[o], shared.at[ibufs[o]], semcs[o]).wait()
                    pltpu.make_async_copy(
                        sbufs[o], shared_s.at[d16bufs[o]], semss[o]).wait()

                @pl.when(j + 1 < KJ2)
                def _():
                    nbase = (wid * KJ2 + j + 1) * CHS
                    pltpu.async_copy(dstr.at[wid, j + 1], ibufs[o], semrs[o])
                    pltpu.async_copy(num_hbm.at[pl.ds(nbase, CHS)],
                                     cbufs[o], semrs[o])
                    pltpu.async_copy(srow_hbm.at[pl.ds(nbase, CHS)],
                                     sbufs[o], semrs[o])

                pltpu.make_async_copy(dstr.at[wid, j], ibufs[p],
                                      semrs[p]).wait()
                pltpu.make_async_copy(num_hbm.at[pl.ds(base, CHS)],
                                      cbufs[p], semrs[p]).wait()
                pltpu.make_async_copy(srow_hbm.at[pl.ds(base, CHS)],
                                      sbufs[p], semrs[p]).wait()
                for g in range(CHS // 16):
                    dvec = ibufs[p][pl.ds(g * 16, 16)]
                    d16bufs[p][pl.ds(g * 16, 16)] = \
                        lax.shift_right_logical(dvec, 4)
                pltpu.async_copy(cbufs[p], shared.at[ibufs[p]],
                                 semcs[p], add=True)
                pltpu.async_copy(sbufs[p], shared_s.at[d16bufs[p]],
                                 semss[p], add=True)
        return carry

    lax.fori_loop(0, (KJ2 + 1) // 2, outer, 0)
    pl_ = (KJ2 - 1) % 2
    pltpu.make_async_copy(cbufs[pl_], shared.at[ibufs[pl_]],
                          semcs[pl_]).wait()
    pltpu.make_async_copy(sbufs[pl_], shared_s.at[d16bufs[pl_]],
                          semss[pl_]).wait()
    plsc.subcore_barrier()
    pltpu.sync_copy(shared.at[pl.ds(sid * ZR, ZR)],
                    accv_hbm.at[cid, pl.ds(sid * ZR, ZR)])
    pltpu.sync_copy(shared_s.at[pl.ds(sid * ZR16, ZR16)],
                    accs_hbm.at[cid, pl.ds(sid * ZR16, ZR16)])


def _final_body(D, accv_ref, s_ref, h_ref, ow_ref, ob_ref, erep_ref,
                pa_ref, out_ref):
    ovec = accv_ref[0] + accv_ref[1]
    s8 = s_ref[0] + s_ref[1]
    srep = jnp.dot(s8, erep_ref[...], preferred_element_type=jnp.float32)
    att = ovec / jnp.maximum(srep, 1e-20)
    y = jnp.dot(att, ow_ref[...], preferred_element_type=jnp.float32) \
        + ob_ref[...] + h_ref[...]
    a = pa_ref[0, 0]
    out_ref[...] = jnp.where(y >= 0, y, a * y)


def kernel(h, edge_index, edge_feat, W_q, W_k, W_v, eW1, eb1, eW2, eb2,
           out_W, out_b, prelu_a):
    N, D = h.shape
    E = edge_index.shape[1]
    DE = edge_feat.shape[1]
    H = eW2.shape[1]
    hd = D // H
    f32 = jnp.float32

    Wq = W_q * (1.0 / math.sqrt(hd))
    Wkv = jnp.concatenate([W_k, W_v], axis=1)
    eye = jnp.eye(H, dtype=f32)
    erep = jnp.repeat(eye, hd, axis=1)
    esum = erep.T
    s1 = jnp.repeat(jnp.eye(16, dtype=f32), H, axis=1)
    s2 = jnp.tile(eye, (1, 16))
    b1 = eb1.reshape(1, D)
    b2 = eb2.reshape(1, H)
    ob = out_b.reshape(1, D)
    pa = prelu_a.reshape(1, 1)

    KJ = -(-E // (NW * CH))
    Epad = NW * CH * KJ
    src = edge_index[0]
    dst = edge_index[1]
    pad = Epad - E
    dstp = jnp.concatenate([dst, jnp.zeros((pad,), jnp.int32)])
    srcp = jnp.concatenate([src, jnp.zeros((pad,), jnp.int32)])
    dstr = dstp.reshape(NW, KJ, CH)
    srcr = srcp.reshape(NW, KJ, CH)
    dst2 = dstp.reshape(Epad, 1)
    efp = jnp.concatenate([edge_feat, jnp.zeros((pad, DE), f32)])
    Np = -(-N // (NS * 8)) * (NS * 8)
    ZR = Np // NS
    NP16 = -(-(-(-Np // 16)) // (NS * 8)) * (NS * 8)
    ZR16 = NP16 // NS
    zeros = jnp.zeros((ZR, D), f32)

    BN = 1000 if N % 1000 == 0 else N
    q, kvp = pl.pallas_call(
        functools.partial(_qkv_body, D),
        grid=(N // BN,),
        in_specs=[
            pl.BlockSpec((BN, D), lambda i: (i, 0)),
            pl.BlockSpec((D, D), lambda i: (0, 0)),
            pl.BlockSpec((D, 2 * D), lambda i: (0, 0)),
        ],
        out_specs=[
            pl.BlockSpec((BN, D), lambda i: (i, 0)),
            pl.BlockSpec((BN, D), lambda i: (i, 0)),
        ],
        out_shape=[
            jax.ShapeDtypeStruct((N, D), f32),
            jax.ShapeDtypeStruct((N, D), f32),
        ],
    )(h, Wq, Wkv)

    mesh = plsc.VectorSubcoreMesh(core_axis_name="c", subcore_axis_name="s")
    gather = pl.kernel(
        functools.partial(_gather_body, KJ),
        out_type=(
            jax.ShapeDtypeStruct((Epad, D), f32),
            jax.ShapeDtypeStruct((Epad, D), f32),
        ),
        mesh=mesh,
        scratch_types=(
            pltpu.VMEM((KJ, CH), jnp.int32),
            pltpu.VMEM((KJ, CH), jnp.int32),
            pltpu.VMEM((CH, D), f32),
            pltpu.VMEM((CH, D), f32),
            pltpu.VMEM((CH, D), f32),
            pltpu.VMEM((CH, D), f32),
            pltpu.SemaphoreType.DMA,
            pltpu.SemaphoreType.DMA,
            pltpu.SemaphoreType.DMA,
            pltpu.SemaphoreType.DMA,
        ),
    )
    qd, kvs = gather(q, kvp, dstr, srcr)

    BE = 1024
    nbe = -(-Epad // BE)
    num, srow = pl.pallas_call(
        functools.partial(_edge_body, E, BE, D, H),
        grid=(nbe,),
        in_specs=[
            pl.BlockSpec((BE, D), lambda i: (i, 0)),
            pl.BlockSpec((BE, D), lambda i: (i, 0)),
            pl.BlockSpec((BE, DE), lambda i: (i, 0)),
            pl.BlockSpec((BE, 1), lambda i: (i, 0)),
            pl.BlockSpec((DE, D), lambda i: (0, 0)),
            pl.BlockSpec((1, D), lambda i: (0, 0)),
            pl.BlockSpec((D, H), lambda i: (0, 0)),
            pl.BlockSpec((1, H), lambda i: (0, 0)),
            pl.BlockSpec((D, H), lambda i: (0, 0)),
            pl.BlockSpec((H, D), lambda i: (0, 0)),
            pl.BlockSpec((16, D), lambda i: (0, 0)),
            pl.BlockSpec((H, D), lambda i: (0, 0)),
        ],
        out_specs=[
            pl.BlockSpec((BE, D), lambda i: (i, 0)),
            pl.BlockSpec((BE, D), lambda i: (i, 0)),
        ],
        out_shape=[
            jax.ShapeDtypeStruct((Epad, D), f32),
            jax.ShapeDtypeStruct((Epad, D), f32),
        ],
    )(qd, kvs, efp, dst2, eW1, b1, eW2, b2, esum, erep, s1, s2)

    KJ2 = Epad // (NW * CHS)
    dstr2 = dstp.reshape(NW, KJ2, CHS)
    scatter = pl.kernel(
        functools.partial(_scatter_body, KJ2, ZR, ZR16),
        out_type=(
            jax.ShapeDtypeStruct((NC, Np, D), f32),
            jax.ShapeDtypeStruct((NC, NP16, D), f32),
        ),
        mesh=mesh,
        scratch_types=(
            pltpu.VMEM_SHARED((Np, D), f32),
            pltpu.VMEM_SHARED((NP16, D), f32),
            pltpu.VMEM((CHS,), jnp.int32),
            pltpu.VMEM((CHS,), jnp.int32),
            pltpu.VMEM((CHS,), jnp.int32),
            pltpu.VMEM((CHS,), jnp.int32),
            pltpu.VMEM((CHS, D), f32),
            pltpu.VMEM((CHS, D), f32),
            pltpu.VMEM((CHS, D), f32),
            pltpu.VMEM((CHS, D), f32),
            pltpu.SemaphoreType.DMA,
            pltpu.SemaphoreType.DMA,
            pltpu.SemaphoreType.DMA,
            pltpu.SemaphoreType.DMA,
            pltpu.SemaphoreType.DMA,
            pltpu.SemaphoreType.DMA,
        ),
    )
    accv, accs = scatter(num, srow, dstr2, zeros)
    s3 = accs.reshape(NC, NP16 * 16, H)

    h_out = pl.pallas_call(
        functools.partial(_final_body, D),
        grid=(N // BN,),
        in_specs=[
            pl.BlockSpec((NC, BN, D), lambda i: (0, i, 0)),
            pl.BlockSpec((NC, BN, H), lambda i: (0, i, 0)),
            pl.BlockSpec((BN, D), lambda i: (i, 0)),
            pl.BlockSpec((D, D), lambda i: (0, 0)),
            pl.BlockSpec((1, D), lambda i: (0, 0)),
            pl.BlockSpec((H, D), lambda i: (0, 0)),
            pl.BlockSpec((1, 1), lambda i: (0, 0)),
        ],
        out_specs=pl.BlockSpec((BN, D), lambda i: (i, 0)),
        out_shape=jax.ShapeDtypeStruct((N, D), f32),
    )(accv, s3, h, out_W, ob, erep, pa)
    return h_out


# speedup vs baseline: 2.2758x; 1.1041x over previous
import functools
import math

import jax
import jax.numpy as jnp
from jax import lax
from jax.experimental import pallas as pl
from jax.experimental.pallas import tpu as pltpu
from jax.experimental.pallas import tpu_sc as plsc

NC = 2
NS = 16
NW = NC * NS
CH = 128


def _qkv_body(D, h_ref, wq_ref, wkv_ref, q_ref, kv_ref):
    hb = h_ref[...]
    q_ref[...] = jnp.dot(hb, wq_ref[...], preferred_element_type=jnp.float32)
    kvf = jnp.dot(hb, wkv_ref[...], preferred_element_type=jnp.float32)
    n = kvf.shape[0]
    k = kvf[:, :D].astype(jnp.bfloat16)
    v = kvf[:, D:].astype(jnp.bfloat16)
    kvi = jnp.concatenate([k[:, None, :], v[:, None, :]],
                          axis=1).reshape(2 * n, D)
    kv_ref.bitcast(jnp.bfloat16)[...] = kvi


def _edge_body(E, E_off, BE, D, H, qd_ref, kvs_ref, ef_ref, d2_ref, w1_ref,
               b1_ref, w2_ref, b2_ref, esum_ref, erep_ref, s1_ref, s2_ref,
               num_ref, srow_ref):
    i = pl.program_id(0)
    qd = qd_ref[...]
    kvv = kvs_ref.bitcast(jnp.bfloat16)[...]
    kv3 = kvv.reshape(BE, 2, D)
    ks = kv3[:, 0, :].astype(jnp.float32)
    vs = kv3[:, 1, :].astype(jnp.float32)
    hid = jnp.maximum(
        jnp.dot(ef_ref[...], w1_ref[...], preferred_element_type=jnp.float32)
        + b1_ref[...], 0.0)
    bias = jnp.dot(hid, w2_ref[...], preferred_element_type=jnp.float32) \
        + b2_ref[...]
    score = jnp.dot(qd * ks, esum_ref[...],
                    preferred_element_type=jnp.float32)
    ex = jnp.exp(score + bias)
    row = E_off + i * BE + lax.broadcasted_iota(jnp.int32, (BE, 1), 0)
    ex = ex * (row < E).astype(jnp.float32)
    exrep = jnp.dot(ex, erep_ref[...],
                    preferred_element_type=jnp.float32)
    num_ref[...] = exrep * vs
    m16 = jnp.bitwise_and(d2_ref[...], 15)
    oh = (m16 == lax.broadcasted_iota(jnp.int32, (BE, 16), 1))
    oh = oh.astype(jnp.float32)
    srow_ref[...] = (
        jnp.dot(oh, s1_ref[...], preferred_element_type=jnp.float32)
        * jnp.dot(ex, s2_ref[...], preferred_element_type=jnp.float32))


def _gather_body(KJ, q_hbm, kv_hbm, dstr, srcr, qd_hbm, kvs_hbm,
                 dloc, sloc, qbuf0, kvbuf0, qbuf1, kvbuf1,
                 semg0, semg1, semw0, semw1):
    cid = lax.axis_index("c")
    sid = lax.axis_index("s")
    wid = sid * NC + cid
    pltpu.sync_copy(dstr.at[wid], dloc)
    pltpu.sync_copy(srcr.at[wid], sloc)
    qbufs = (qbuf0, qbuf1)
    kvbufs = (kvbuf0, kvbuf1)
    semgs = (semg0, semg1)
    semws = (semw0, semw1)

    pltpu.async_copy(q_hbm.at[dloc.at[0]], qbuf0, semg0)
    pltpu.async_copy(kv_hbm.at[sloc.at[0]], kvbuf0, semg0)

    def outer(jj, carry):
        for b in range(2):
            j = jj * 2 + b
            p = b
            o = 1 - b

            @pl.when(j < KJ)
            def _():
                base = (wid * KJ + j) * CH
                @pl.when(j >= 1)
                def _():
                    pltpu.make_async_copy(
                        qbufs[o], qd_hbm.at[pl.ds(base, CH)], semws[o]).wait()
                    pltpu.make_async_copy(
                        kvbufs[o], kvs_hbm.at[pl.ds(base, CH)],
                        semws[o]).wait()

                @pl.when(j + 1 < KJ)
                def _():
                    pltpu.async_copy(q_hbm.at[dloc.at[j + 1]], qbufs[o],
                                     semgs[o])
                    pltpu.async_copy(kv_hbm.at[sloc.at[j + 1]], kvbufs[o],
                                     semgs[o])

                pltpu.make_async_copy(q_hbm.at[dloc.at[j]], qbufs[p],
                                      semgs[p]).wait()
                pltpu.make_async_copy(kv_hbm.at[sloc.at[j]], kvbufs[p],
                                      semgs[p]).wait()

                @pl.when(j == KJ - 1)
                def _():
                    pltpu.sync_copy(qbufs[p], qd_hbm.at[pl.ds(base, CH)])
                    pltpu.sync_copy(kvbufs[p], kvs_hbm.at[pl.ds(base, CH)])

                @pl.when(j < KJ - 1)
                def _():
                    pltpu.async_copy(qbufs[p], qd_hbm.at[pl.ds(base, CH)],
                                     semws[p])
                    pltpu.async_copy(kvbufs[p], kvs_hbm.at[pl.ds(base, CH)],
                                     semws[p])
        return carry

    lax.fori_loop(0, (KJ + 1) // 2, outer, 0)


CHS = 64


def _scatter_body(KJ2, ZR, ZR16, num_hbm, srow_hbm, dstr, accv_in, accs_in,
                  accv_hbm, accs_hbm, shared, shared_s,
                  ibuf0, ibuf1, d16buf0, d16buf1, cbuf0, cbuf1, sbuf0, sbuf1,
                  semr0, semr1, semc0, semc1, sems0, sems1):
    cid = lax.axis_index("c")
    sid = lax.axis_index("s")
    wid = sid * NC + cid
    pltpu.sync_copy(accv_in.at[cid, pl.ds(sid * ZR, ZR)],
                    shared.at[pl.ds(sid * ZR, ZR)])
    pltpu.sync_copy(accs_in.at[cid, pl.ds(sid * ZR16, ZR16)],
                    shared_s.at[pl.ds(sid * ZR16, ZR16)])
    plsc.subcore_barrier()
    ibufs = (ibuf0, ibuf1)
    d16bufs = (d16buf0, d16buf1)
    cbufs = (cbuf0, cbuf1)
    sbufs = (sbuf0, sbuf1)
    semrs = (semr0, semr1)
    semcs = (semc0, semc1)
    semss = (sems0, sems1)

    base0 = wid * KJ2 * CHS
    pltpu.async_copy(dstr.at[wid, 0], ibuf0, semr0)
    pltpu.async_copy(num_hbm.at[pl.ds(base0, CHS)], cbuf0, semr0)
    pltpu.async_copy(srow_hbm.at[pl.ds(base0, CHS)], sbuf0, semr0)

    def outer(jj, carry):
        for b in range(2):
            j = jj * 2 + b
            p = b
            o = 1 - b

            @pl.when(j < KJ2)
            def _():
                base = (wid * KJ2 + j) * CHS

                @pl.when(j >= 1)
                def _():
                    pltpu.make_async_copy(
                        cbufs[o], shared.at[ibufs[o]], semcs[o]).wait()
                    pltpu.make_async_copy(
                        sbufs[o], shared_s.at[d16bufs[o]], semss[o]).wait()

                @pl.when(j + 1 < KJ2)
                def _():
                    nbase = (wid * KJ2 + j + 1) * CHS
                    pltpu.async_copy(dstr.at[wid, j + 1], ibufs[o], semrs[o])
                    pltpu.async_copy(num_hbm.at[pl.ds(nbase, CHS)],
                                     cbufs[o], semrs[o])
                    pltpu.async_copy(srow_hbm.at[pl.ds(nbase, CHS)],
                                     sbufs[o], semrs[o])

                pltpu.make_async_copy(dstr.at[wid, j], ibufs[p],
                                      semrs[p]).wait()
                pltpu.make_async_copy(num_hbm.at[pl.ds(base, CHS)],
                                      cbufs[p], semrs[p]).wait()
                pltpu.make_async_copy(srow_hbm.at[pl.ds(base, CHS)],
                                      sbufs[p], semrs[p]).wait()
                for g in range(CHS // 16):
                    dvec = ibufs[p][pl.ds(g * 16, 16)]
                    d16bufs[p][pl.ds(g * 16, 16)] = \
                        lax.shift_right_logical(dvec, 4)
                pltpu.async_copy(cbufs[p], shared.at[ibufs[p]],
                                 semcs[p], add=True)
                pltpu.async_copy(sbufs[p], shared_s.at[d16bufs[p]],
                                 semss[p], add=True)
        return carry

    lax.fori_loop(0, (KJ2 + 1) // 2, outer, 0)
    pl_ = (KJ2 - 1) % 2
    pltpu.make_async_copy(cbufs[pl_], shared.at[ibufs[pl_]],
                          semcs[pl_]).wait()
    pltpu.make_async_copy(sbufs[pl_], shared_s.at[d16bufs[pl_]],
                          semss[pl_]).wait()
    plsc.subcore_barrier()
    pltpu.sync_copy(shared.at[pl.ds(sid * ZR, ZR)],
                    accv_hbm.at[cid, pl.ds(sid * ZR, ZR)])
    pltpu.sync_copy(shared_s.at[pl.ds(sid * ZR16, ZR16)],
                    accs_hbm.at[cid, pl.ds(sid * ZR16, ZR16)])


def _final_body(D, accv_ref, s_ref, h_ref, ow_ref, ob_ref, erep_ref,
                pa_ref, out_ref):
    ovec = accv_ref[0] + accv_ref[1]
    s8 = s_ref[0] + s_ref[1]
    srep = jnp.dot(s8, erep_ref[...], preferred_element_type=jnp.float32)
    att = ovec / jnp.maximum(srep, 1e-20)
    y = jnp.dot(att, ow_ref[...], preferred_element_type=jnp.float32) \
        + ob_ref[...] + h_ref[...]
    a = pa_ref[0, 0]
    out_ref[...] = jnp.where(y >= 0, y, a * y)


def kernel(h, edge_index, edge_feat, W_q, W_k, W_v, eW1, eb1, eW2, eb2,
           out_W, out_b, prelu_a):
    N, D = h.shape
    E = edge_index.shape[1]
    DE = edge_feat.shape[1]
    H = eW2.shape[1]
    hd = D // H
    f32 = jnp.float32

    Wq = W_q * (1.0 / math.sqrt(hd))
    Wkv = jnp.concatenate([W_k, W_v], axis=1)
    eye = jnp.eye(H, dtype=f32)
    erep = jnp.repeat(eye, hd, axis=1)
    esum = erep.T
    s1 = jnp.repeat(jnp.eye(16, dtype=f32), H, axis=1)
    s2 = jnp.tile(eye, (1, 16))
    b1 = eb1.reshape(1, D)
    b2 = eb2.reshape(1, H)
    ob = out_b.reshape(1, D)
    pa = prelu_a.reshape(1, 1)

    KJ = -(-E // (NW * CH))
    Epad = NW * CH * KJ
    src = edge_index[0]
    dst = edge_index[1]
    pad = Epad - E
    dstp = jnp.concatenate([dst, jnp.zeros((pad,), jnp.int32)])
    srcp = jnp.concatenate([src, jnp.zeros((pad,), jnp.int32)])
    dstr = dstp.reshape(NW, KJ, CH)
    srcr = srcp.reshape(NW, KJ, CH)
    dst2 = dstp.reshape(Epad, 1)
    efp = jnp.concatenate([edge_feat, jnp.zeros((pad, DE), f32)])
    Np = -(-N // (NS * 8)) * (NS * 8)
    ZR = Np // NS
    NP16 = -(-(-(-Np // 16)) // (NS * 8)) * (NS * 8)
    ZR16 = NP16 // NS

    BN = 1000 if N % 1000 == 0 else N
    q, kvp = pl.pallas_call(
        functools.partial(_qkv_body, D),
        grid=(N // BN,),
        in_specs=[
            pl.BlockSpec((BN, D), lambda i: (i, 0)),
            pl.BlockSpec((D, D), lambda i: (0, 0)),
            pl.BlockSpec((D, 2 * D), lambda i: (0, 0)),
        ],
        out_specs=[
            pl.BlockSpec((BN, D), lambda i: (i, 0)),
            pl.BlockSpec((BN, D), lambda i: (i, 0)),
        ],
        out_shape=[
            jax.ShapeDtypeStruct((N, D), f32),
            jax.ShapeDtypeStruct((N, D), f32),
        ],
    )(h, Wq, Wkv)

    mesh = plsc.VectorSubcoreMesh(core_axis_name="c", subcore_axis_name="s")
    BE = 1024
    NSL = 2
    kj_split = [KJ // NSL + (1 if s < KJ % NSL else 0) for s in range(NSL)]
    accv = jnp.zeros((NC, Np, D), f32)
    accs = jnp.zeros((NC, NP16, D), f32)
    e_off = 0
    for KJs in kj_split:
        Epads = NW * CH * KJs
        sl = slice(e_off, e_off + Epads)
        dstr = dstp[sl].reshape(NW, KJs, CH)
        srcr = srcp[sl].reshape(NW, KJs, CH)
        dst2s = dstp[sl].reshape(Epads, 1)
        efps = efp[sl]
        gather = pl.kernel(
            functools.partial(_gather_body, KJs),
            out_type=(
                jax.ShapeDtypeStruct((Epads, D), f32),
                jax.ShapeDtypeStruct((Epads, D), f32),
            ),
            mesh=mesh,
            scratch_types=(
                pltpu.VMEM((KJs, CH), jnp.int32),
                pltpu.VMEM((KJs, CH), jnp.int32),
                pltpu.VMEM((CH, D), f32),
                pltpu.VMEM((CH, D), f32),
                pltpu.VMEM((CH, D), f32),
                pltpu.VMEM((CH, D), f32),
                pltpu.SemaphoreType.DMA,
                pltpu.SemaphoreType.DMA,
                pltpu.SemaphoreType.DMA,
                pltpu.SemaphoreType.DMA,
            ),
        )
        qd, kvs = gather(q, kvp, dstr, srcr)

        nbe = Epads // BE
        num, srow = pl.pallas_call(
            functools.partial(_edge_body, E, e_off, BE, D, H),
            grid=(nbe,),
            in_specs=[
                pl.BlockSpec((BE, D), lambda i: (i, 0)),
                pl.BlockSpec((BE, D), lambda i: (i, 0)),
                pl.BlockSpec((BE, DE), lambda i: (i, 0)),
                pl.BlockSpec((BE, 1), lambda i: (i, 0)),
                pl.BlockSpec((DE, D), lambda i: (0, 0)),
                pl.BlockSpec((1, D), lambda i: (0, 0)),
                pl.BlockSpec((D, H), lambda i: (0, 0)),
                pl.BlockSpec((1, H), lambda i: (0, 0)),
                pl.BlockSpec((D, H), lambda i: (0, 0)),
                pl.BlockSpec((H, D), lambda i: (0, 0)),
                pl.BlockSpec((16, D), lambda i: (0, 0)),
                pl.BlockSpec((H, D), lambda i: (0, 0)),
            ],
            out_specs=[
                pl.BlockSpec((BE, D), lambda i: (i, 0)),
                pl.BlockSpec((BE, D), lambda i: (i, 0)),
            ],
            out_shape=[
                jax.ShapeDtypeStruct((Epads, D), f32),
                jax.ShapeDtypeStruct((Epads, D), f32),
            ],
        )(qd, kvs, efps, dst2s, eW1, b1, eW2, b2, esum, erep, s1, s2)

        KJ2 = Epads // (NW * CHS)
        dstr2 = dstp[sl].reshape(NW, KJ2, CHS)
        scatter = pl.kernel(
            functools.partial(_scatter_body, KJ2, ZR, ZR16),
            out_type=(
                jax.ShapeDtypeStruct((NC, Np, D), f32),
                jax.ShapeDtypeStruct((NC, NP16, D), f32),
            ),
            mesh=mesh,
            scratch_types=(
                pltpu.VMEM_SHARED((Np, D), f32),
                pltpu.VMEM_SHARED((NP16, D), f32),
                pltpu.VMEM((CHS,), jnp.int32),
                pltpu.VMEM((CHS,), jnp.int32),
                pltpu.VMEM((CHS,), jnp.int32),
                pltpu.VMEM((CHS,), jnp.int32),
                pltpu.VMEM((CHS, D), f32),
                pltpu.VMEM((CHS, D), f32),
                pltpu.VMEM((CHS, D), f32),
                pltpu.VMEM((CHS, D), f32),
                pltpu.SemaphoreType.DMA,
                pltpu.SemaphoreType.DMA,
                pltpu.SemaphoreType.DMA,
                pltpu.SemaphoreType.DMA,
                pltpu.SemaphoreType.DMA,
                pltpu.SemaphoreType.DMA,
            ),
        )
        accv, accs = scatter(num, srow, dstr2, accv, accs)
        e_off += Epads
    s3 = accs.reshape(NC, NP16 * 16, H)

    h_out = pl.pallas_call(
        functools.partial(_final_body, D),
        grid=(N // BN,),
        in_specs=[
            pl.BlockSpec((NC, BN, D), lambda i: (0, i, 0)),
            pl.BlockSpec((NC, BN, H), lambda i: (0, i, 0)),
            pl.BlockSpec((BN, D), lambda i: (i, 0)),
            pl.BlockSpec((D, D), lambda i: (0, 0)),
            pl.BlockSpec((1, D), lambda i: (0, 0)),
            pl.BlockSpec((H, D), lambda i: (0, 0)),
            pl.BlockSpec((1, 1), lambda i: (0, 0)),
        ],
        out_specs=pl.BlockSpec((BN, D), lambda i: (i, 0)),
        out_shape=jax.ShapeDtypeStruct((N, D), f32),
    )(accv, s3, h, out_W, ob, erep, pa)
    return h_out


# speedup vs baseline: 2.2916x; 1.0069x over previous
import functools
import math

import jax
import jax.numpy as jnp
from jax import lax
from jax.experimental import pallas as pl
from jax.experimental.pallas import tpu as pltpu
from jax.experimental.pallas import tpu_sc as plsc

NC = 2
NS = 16
NW = NC * NS
CH = 128


def _qkv_body(D, h_ref, wq_ref, wkv_ref, q_ref, kv_ref):
    hb = h_ref[...]
    q_ref[...] = jnp.dot(hb, wq_ref[...], preferred_element_type=jnp.float32)
    kvf = jnp.dot(hb, wkv_ref[...], preferred_element_type=jnp.float32)
    n = kvf.shape[0]
    k = kvf[:, :D].astype(jnp.bfloat16)
    v = kvf[:, D:].astype(jnp.bfloat16)
    kvi = jnp.concatenate([k[:, None, :], v[:, None, :]],
                          axis=1).reshape(2 * n, D)
    kv_ref.bitcast(jnp.bfloat16)[...] = kvi


def _edge_body(E, E_off, BE, D, H, qd_ref, kvs_ref, ef_ref, d2_ref, w1_ref,
               b1_ref, w2_ref, b2_ref, esum_ref, erep_ref, s1_ref, s2_ref,
               num_ref, srow_ref):
    i = pl.program_id(0)
    qd = qd_ref[...]
    kvv = kvs_ref.bitcast(jnp.bfloat16)[...]
    kv3 = kvv.reshape(BE, 2, D)
    ks = kv3[:, 0, :].astype(jnp.float32)
    vs = kv3[:, 1, :].astype(jnp.float32)
    hid = jnp.maximum(
        jnp.dot(ef_ref[...], w1_ref[...], preferred_element_type=jnp.float32)
        + b1_ref[...], 0.0)
    bias = jnp.dot(hid, w2_ref[...], preferred_element_type=jnp.float32) \
        + b2_ref[...]
    score = jnp.dot(qd * ks, esum_ref[...],
                    preferred_element_type=jnp.float32)
    ex = jnp.exp(score + bias)
    row = E_off + i * BE + lax.broadcasted_iota(jnp.int32, (BE, 1), 0)
    ex = ex * (row < E).astype(jnp.float32)
    exrep = jnp.dot(ex, erep_ref[...],
                    preferred_element_type=jnp.float32)
    num_ref[...] = exrep * vs
    m16 = jnp.bitwise_and(d2_ref[...], 15)
    oh = (m16 == lax.broadcasted_iota(jnp.int32, (BE, 16), 1))
    oh = oh.astype(jnp.float32)
    srow_ref[...] = (
        jnp.dot(oh, s1_ref[...], preferred_element_type=jnp.float32)
        * jnp.dot(ex, s2_ref[...], preferred_element_type=jnp.float32))


def _gather_body(KJ, q_hbm, kv_hbm, dstr, srcr, qd_hbm, kvs_hbm,
                 dloc, sloc, qbuf0, kvbuf0, qbuf1, kvbuf1,
                 semg0, semg1, semw0, semw1):
    cid = lax.axis_index("c")
    sid = lax.axis_index("s")
    wid = sid * NC + cid
    pltpu.sync_copy(dstr.at[wid], dloc)
    pltpu.sync_copy(srcr.at[wid], sloc)
    qbufs = (qbuf0, qbuf1)
    kvbufs = (kvbuf0, kvbuf1)
    semgs = (semg0, semg1)
    semws = (semw0, semw1)

    pltpu.async_copy(q_hbm.at[dloc.at[0]], qbuf0, semg0)
    pltpu.async_copy(kv_hbm.at[sloc.at[0]], kvbuf0, semg0)

    def outer(jj, carry):
        for b in range(2):
            j = jj * 2 + b
            p = b
            o = 1 - b

            @pl.when(j < KJ)
            def _():
                base = (wid * KJ + j) * CH
                @pl.when(j >= 1)
                def _():
                    pltpu.make_async_copy(
                        qbufs[o], qd_hbm.at[pl.ds(base, CH)], semws[o]).wait()
                    pltpu.make_async_copy(
                        kvbufs[o], kvs_hbm.at[pl.ds(base, CH)],
                        semws[o]).wait()

                @pl.when(j + 1 < KJ)
                def _():
                    pltpu.async_copy(q_hbm.at[dloc.at[j + 1]], qbufs[o],
                                     semgs[o])
                    pltpu.async_copy(kv_hbm.at[sloc.at[j + 1]], kvbufs[o],
                                     semgs[o])

                pltpu.make_async_copy(q_hbm.at[dloc.at[j]], qbufs[p],
                                      semgs[p]).wait()
                pltpu.make_async_copy(kv_hbm.at[sloc.at[j]], kvbufs[p],
                                      semgs[p]).wait()

                @pl.when(j == KJ - 1)
                def _():
                    pltpu.sync_copy(qbufs[p], qd_hbm.at[pl.ds(base, CH)])
                    pltpu.sync_copy(kvbufs[p], kvs_hbm.at[pl.ds(base, CH)])

                @pl.when(j < KJ - 1)
                def _():
                    pltpu.async_copy(qbufs[p], qd_hbm.at[pl.ds(base, CH)],
                                     semws[p])
                    pltpu.async_copy(kvbufs[p], kvs_hbm.at[pl.ds(base, CH)],
                                     semws[p])
        return carry

    lax.fori_loop(0, (KJ + 1) // 2, outer, 0)


CHS = 64


def _scatter_body(KJ2, ZR, ZR16, num_hbm, srow_hbm, dstr, accv_in, accs_in,
                  accv_hbm, accs_hbm, shared, shared_s,
                  ibuf0, ibuf1, d16buf0, d16buf1, cbuf0, cbuf1, sbuf0, sbuf1,
                  semr0, semr1, semc0, semc1, sems0, sems1):
    cid = lax.axis_index("c")
    sid = lax.axis_index("s")
    wid = sid * NC + cid
    pltpu.sync_copy(accv_in.at[cid, pl.ds(sid * ZR, ZR)],
                    shared.at[pl.ds(sid * ZR, ZR)])
    pltpu.sync_copy(accs_in.at[cid, pl.ds(sid * ZR16, ZR16)],
                    shared_s.at[pl.ds(sid * ZR16, ZR16)])
    plsc.subcore_barrier()
    ibufs = (ibuf0, ibuf1)
    d16bufs = (d16buf0, d16buf1)
    cbufs = (cbuf0, cbuf1)
    sbufs = (sbuf0, sbuf1)
    semrs = (semr0, semr1)
    semcs = (semc0, semc1)
    semss = (sems0, sems1)

    base0 = wid * KJ2 * CHS
    pltpu.async_copy(dstr.at[wid, 0], ibuf0, semr0)
    pltpu.async_copy(num_hbm.at[pl.ds(base0, CHS)], cbuf0, semr0)
    pltpu.async_copy(srow_hbm.at[pl.ds(base0, CHS)], sbuf0, semr0)

    def outer(jj, carry):
        for b in range(2):
            j = jj * 2 + b
            p = b
            o = 1 - b

            @pl.when(j < KJ2)
            def _():
                base = (wid * KJ2 + j) * CHS

                @pl.when(j >= 1)
                def _():
                    pltpu.make_async_copy(
                        cbufs[o], shared.at[ibufs[o]], semcs[o]).wait()
                    pltpu.make_async_copy(
                        sbufs[o], shared_s.at[d16bufs[o]], semss[o]).wait()

                @pl.when(j + 1 < KJ2)
                def _():
                    nbase = (wid * KJ2 + j + 1) * CHS
                    pltpu.async_copy(dstr.at[wid, j + 1], ibufs[o], semrs[o])
                    pltpu.async_copy(num_hbm.at[pl.ds(nbase, CHS)],
                                     cbufs[o], semrs[o])
                    pltpu.async_copy(srow_hbm.at[pl.ds(nbase, CHS)],
                                     sbufs[o], semrs[o])

                pltpu.make_async_copy(dstr.at[wid, j], ibufs[p],
                                      semrs[p]).wait()
                pltpu.make_async_copy(num_hbm.at[pl.ds(base, CHS)],
                                      cbufs[p], semrs[p]).wait()
                pltpu.make_async_copy(srow_hbm.at[pl.ds(base, CHS)],
                                      sbufs[p], semrs[p]).wait()
                for g in range(CHS // 16):
                    dvec = ibufs[p][pl.ds(g * 16, 16)]
                    d16bufs[p][pl.ds(g * 16, 16)] = \
                        lax.shift_right_logical(dvec, 4)
                pltpu.async_copy(cbufs[p], shared.at[ibufs[p]],
                                 semcs[p], add=True)
                pltpu.async_copy(sbufs[p], shared_s.at[d16bufs[p]],
                                 semss[p], add=True)
        return carry

    lax.fori_loop(0, (KJ2 + 1) // 2, outer, 0)
    pl_ = (KJ2 - 1) % 2
    pltpu.make_async_copy(cbufs[pl_], shared.at[ibufs[pl_]],
                          semcs[pl_]).wait()
    pltpu.make_async_copy(sbufs[pl_], shared_s.at[d16bufs[pl_]],
                          semss[pl_]).wait()
    plsc.subcore_barrier()
    pltpu.sync_copy(shared.at[pl.ds(sid * ZR, ZR)],
                    accv_hbm.at[cid, pl.ds(sid * ZR, ZR)])
    pltpu.sync_copy(shared_s.at[pl.ds(sid * ZR16, ZR16)],
                    accs_hbm.at[cid, pl.ds(sid * ZR16, ZR16)])


def _final_body(D, accv_ref, s_ref, h_ref, ow_ref, ob_ref, erep_ref,
                pa_ref, out_ref):
    ovec = accv_ref[0] + accv_ref[1]
    s8 = s_ref[0] + s_ref[1]
    srep = jnp.dot(s8, erep_ref[...], preferred_element_type=jnp.float32)
    att = ovec / jnp.maximum(srep, 1e-20)
    y = jnp.dot(att, ow_ref[...], preferred_element_type=jnp.float32) \
        + ob_ref[...] + h_ref[...]
    a = pa_ref[0, 0]
    out_ref[...] = jnp.where(y >= 0, y, a * y)


def kernel(h, edge_index, edge_feat, W_q, W_k, W_v, eW1, eb1, eW2, eb2,
           out_W, out_b, prelu_a):
    N, D = h.shape
    E = edge_index.shape[1]
    DE = edge_feat.shape[1]
    H = eW2.shape[1]
    hd = D // H
    f32 = jnp.float32

    Wq = W_q * (1.0 / math.sqrt(hd))
    Wkv = jnp.concatenate([W_k, W_v], axis=1)
    eye = jnp.eye(H, dtype=f32)
    erep = jnp.repeat(eye, hd, axis=1)
    esum = erep.T
    s1 = jnp.repeat(jnp.eye(16, dtype=f32), H, axis=1)
    s2 = jnp.tile(eye, (1, 16))
    b1 = eb1.reshape(1, D)
    b2 = eb2.reshape(1, H)
    ob = out_b.reshape(1, D)
    pa = prelu_a.reshape(1, 1)

    KJ = -(-E // (NW * CH))
    Epad = NW * CH * KJ
    src = edge_index[0]
    dst = edge_index[1]
    pad = Epad - E
    dstp = jnp.concatenate([dst, jnp.zeros((pad,), jnp.int32)])
    srcp = jnp.concatenate([src, jnp.zeros((pad,), jnp.int32)])
    dstr = dstp.reshape(NW, KJ, CH)
    srcr = srcp.reshape(NW, KJ, CH)
    dst2 = dstp.reshape(Epad, 1)
    efp = jnp.concatenate([edge_feat, jnp.zeros((pad, DE), f32)])
    Np = -(-N // (NS * 8)) * (NS * 8)
    ZR = Np // NS
    NP16 = -(-(-(-Np // 16)) // (NS * 8)) * (NS * 8)
    ZR16 = NP16 // NS

    BN = 1000 if N % 1000 == 0 else N
    q, kvp = pl.pallas_call(
        functools.partial(_qkv_body, D),
        grid=(N // BN,),
        in_specs=[
            pl.BlockSpec((BN, D), lambda i: (i, 0)),
            pl.BlockSpec((D, D), lambda i: (0, 0)),
            pl.BlockSpec((D, 2 * D), lambda i: (0, 0)),
        ],
        out_specs=[
            pl.BlockSpec((BN, D), lambda i: (i, 0)),
            pl.BlockSpec((BN, D), lambda i: (i, 0)),
        ],
        out_shape=[
            jax.ShapeDtypeStruct((N, D), f32),
            jax.ShapeDtypeStruct((N, D), f32),
        ],
    )(h, Wq, Wkv)

    mesh = plsc.VectorSubcoreMesh(core_axis_name="c", subcore_axis_name="s")
    BE = 1024
    NSL = 4
    kj_split = [KJ // NSL + (1 if s < KJ % NSL else 0) for s in range(NSL)]
    accv = jnp.zeros((NC, Np, D), f32)
    accs = jnp.zeros((NC, NP16, D), f32)
    e_off = 0
    for KJs in kj_split:
        Epads = NW * CH * KJs
        sl = slice(e_off, e_off + Epads)
        dstr = dstp[sl].reshape(NW, KJs, CH)
        srcr = srcp[sl].reshape(NW, KJs, CH)
        dst2s = dstp[sl].reshape(Epads, 1)
        efps = efp[sl]
        gather = pl.kernel(
            functools.partial(_gather_body, KJs),
            out_type=(
                jax.ShapeDtypeStruct((Epads, D), f32),
                jax.ShapeDtypeStruct((Epads, D), f32),
            ),
            mesh=mesh,
            scratch_types=(
                pltpu.VMEM((KJs, CH), jnp.int32),
                pltpu.VMEM((KJs, CH), jnp.int32),
                pltpu.VMEM((CH, D), f32),
                pltpu.VMEM((CH, D), f32),
                pltpu.VMEM((CH, D), f32),
                pltpu.VMEM((CH, D), f32),
                pltpu.SemaphoreType.DMA,
                pltpu.SemaphoreType.DMA,
                pltpu.SemaphoreType.DMA,
                pltpu.SemaphoreType.DMA,
            ),
        )
        qd, kvs = gather(q, kvp, dstr, srcr)

        nbe = Epads // BE
        num, srow = pl.pallas_call(
            functools.partial(_edge_body, E, e_off, BE, D, H),
            grid=(nbe,),
            in_specs=[
                pl.BlockSpec((BE, D), lambda i: (i, 0)),
                pl.BlockSpec((BE, D), lambda i: (i, 0)),
                pl.BlockSpec((BE, DE), lambda i: (i, 0)),
                pl.BlockSpec((BE, 1), lambda i: (i, 0)),
                pl.BlockSpec((DE, D), lambda i: (0, 0)),
                pl.BlockSpec((1, D), lambda i: (0, 0)),
                pl.BlockSpec((D, H), lambda i: (0, 0)),
                pl.BlockSpec((1, H), lambda i: (0, 0)),
                pl.BlockSpec((D, H), lambda i: (0, 0)),
                pl.BlockSpec((H, D), lambda i: (0, 0)),
                pl.BlockSpec((16, D), lambda i: (0, 0)),
                pl.BlockSpec((H, D), lambda i: (0, 0)),
            ],
            out_specs=[
                pl.BlockSpec((BE, D), lambda i: (i, 0)),
                pl.BlockSpec((BE, D), lambda i: (i, 0)),
            ],
            out_shape=[
                jax.ShapeDtypeStruct((Epads, D), f32),
                jax.ShapeDtypeStruct((Epads, D), f32),
            ],
        )(qd, kvs, efps, dst2s, eW1, b1, eW2, b2, esum, erep, s1, s2)

        KJ2 = Epads // (NW * CHS)
        dstr2 = dstp[sl].reshape(NW, KJ2, CHS)
        scatter = pl.kernel(
            functools.partial(_scatter_body, KJ2, ZR, ZR16),
            out_type=(
                jax.ShapeDtypeStruct((NC, Np, D), f32),
                jax.ShapeDtypeStruct((NC, NP16, D), f32),
            ),
            mesh=mesh,
            scratch_types=(
                pltpu.VMEM_SHARED((Np, D), f32),
                pltpu.VMEM_SHARED((NP16, D), f32),
                pltpu.VMEM((CHS,), jnp.int32),
                pltpu.VMEM((CHS,), jnp.int32),
                pltpu.VMEM((CHS,), jnp.int32),
                pltpu.VMEM((CHS,), jnp.int32),
                pltpu.VMEM((CHS, D), f32),
                pltpu.VMEM((CHS, D), f32),
                pltpu.VMEM((CHS, D), f32),
                pltpu.VMEM((CHS, D), f32),
                pltpu.SemaphoreType.DMA,
                pltpu.SemaphoreType.DMA,
                pltpu.SemaphoreType.DMA,
                pltpu.SemaphoreType.DMA,
                pltpu.SemaphoreType.DMA,
                pltpu.SemaphoreType.DMA,
            ),
        )
        accv, accs = scatter(num, srow, dstr2, accv, accs)
        e_off += Epads
    s3 = accs.reshape(NC, NP16 * 16, H)

    h_out = pl.pallas_call(
        functools.partial(_final_body, D),
        grid=(N // BN,),
        in_specs=[
            pl.BlockSpec((NC, BN, D), lambda i: (0, i, 0)),
            pl.BlockSpec((NC, BN, H), lambda i: (0, i, 0)),
            pl.BlockSpec((BN, D), lambda i: (i, 0)),
            pl.BlockSpec((D, D), lambda i: (0, 0)),
            pl.BlockSpec((1, D), lambda i: (0, 0)),
            pl.BlockSpec((H, D), lambda i: (0, 0)),
            pl.BlockSpec((1, 1), lambda i: (0, 0)),
        ],
        out_specs=pl.BlockSpec((BN, D), lambda i: (i, 0)),
        out_shape=jax.ShapeDtypeStruct((N, D), f32),
    )(accv, s3, h, out_W, ob, erep, pa)
    return h_out
